# baseline (speedup 1.0000x reference)
import jax
import jax.numpy as jnp
from jax.experimental import pallas as pl
from jax.experimental.pallas import tpu as pltpu

_N, _IN_FEAT, _S_FEAT, _LABELS, _SLOPE = 4096, 512, 256, 2, 0.01
_BR = 512
_GRID = _N // _BR


def _adj_copy(adj_hbm, adj_buf, adj_sem, block, slot):
    return pltpu.make_async_copy(
        adj_hbm.at[pl.ds(block * _BR, _BR), :], adj_buf.at[slot],
        adj_sem.at[slot])


_HC = _N // 2


def _att_copy(att_buf, att_hbm, att_sem, block, half):
    return pltpu.make_async_copy(
        att_buf.at[:, pl.ds(half * _HC, _HC)],
        att_hbm.at[pl.ds(block * _BR, _BR), pl.ds(half * _HC, _HC)],
        att_sem.at[half])


def _gat_kernel(inDoc_ref, W_ref, a1_ref, a2_ref, adj_hbm, clsW_ref, clsb_ref,
                att_hbm, doc_ref, pool_ref, label_ref,
                whb_ref, f1_ref, f2t_ref, adj_buf, adj_sem,
                att_buf, att_sem):
    i = pl.program_id(0)
    slot = jax.lax.rem(i, 2)

    @pl.when(i == 0)
    def _():
        _adj_copy(adj_hbm, adj_buf, adj_sem, 0, 0).start()
        wa1 = jnp.dot(W_ref[...], a1_ref[...],
                      preferred_element_type=jnp.float32)
        wa2 = jnp.dot(W_ref[...], a2_ref[...],
                      preferred_element_type=jnp.float32)
        f1_ref[...] = jnp.dot(inDoc_ref[...], wa1,
                              preferred_element_type=jnp.float32)
        f2t_ref[...] = jax.lax.dot_general(
            wa2, inDoc_ref[...], (((0,), (1,)), ((), ())),
            preferred_element_type=jnp.float32)
        whb_ref[...] = jnp.dot(inDoc_ref[...], W_ref[...],
                               preferred_element_type=jnp.float32
                               ).astype(jnp.bfloat16)

    @pl.when(i + 1 < _GRID)
    def _():
        _adj_copy(adj_hbm, adj_buf, adj_sem, i + 1,
                  jax.lax.rem(i + 1, 2)).start()

    logits = f1_ref[pl.ds(i * _BR, _BR), :] + f2t_ref[...]
    logits = jnp.maximum(logits, _SLOPE * logits)
    q = jnp.exp(logits)

    _adj_copy(adj_hbm, adj_buf, adj_sem, i, slot).wait()
    p = jnp.where(adj_buf[slot] > 0, q, 0.0)
    pb = p.astype(jnp.bfloat16)
    recip = 1.0 / jnp.sum(p, axis=1, keepdims=True)

    @pl.when(i >= 1)
    def _():
        _att_copy(att_buf, att_hbm, att_sem, i - 1, 0).wait()
        _att_copy(att_buf, att_hbm, att_sem, i - 1, 1).wait()

    att_buf[:, : _HC] = p[:, : _HC] * recip
    _att_copy(att_buf, att_hbm, att_sem, i, 0).start()
    att_buf[:, _HC:] = p[:, _HC:] * recip
    _att_copy(att_buf, att_hbm, att_sem, i, 1).start()

    doc = jnp.dot(pb, whb_ref[...],
                  preferred_element_type=jnp.float32) * recip
    doc = jnp.where(doc > 0, doc, jnp.exp(doc) - 1.0)
    doc_ref[...] = doc
    colsum = jnp.sum(doc, axis=0, keepdims=True)

    @pl.when(i == 0)
    def _():
        pool_ref[...] = colsum

    @pl.when(i > 0)
    def _():
        pool_ref[...] += colsum

    @pl.when(i == _GRID - 1)
    def _():
        _att_copy(att_buf, att_hbm, att_sem, i, 0).wait()
        _att_copy(att_buf, att_hbm, att_sem, i, 1).wait()
        pool = pool_ref[...] * (1.0 / _N)
        pool_ref[...] = pool
        cls = jnp.dot(pool, clsW_ref[...],
                      preferred_element_type=jnp.float32) + clsb_ref[...]
        cm = jnp.max(cls, axis=1, keepdims=True)
        cp = jnp.exp(cls - cm)
        label_ref[...] = cp / jnp.sum(cp, axis=1, keepdims=True)


def kernel(inDoc, adj, W, a1, a2, clsW, clsb):
    clsb2 = clsb.reshape(1, _LABELS)

    att, doc, pool, label = pl.pallas_call(
        _gat_kernel,
        grid=(_GRID,),
        in_specs=[
            pl.BlockSpec((_N, _IN_FEAT), lambda i: (0, 0)),
            pl.BlockSpec((_IN_FEAT, _S_FEAT), lambda i: (0, 0)),
            pl.BlockSpec((_S_FEAT, 1), lambda i: (0, 0)),
            pl.BlockSpec((_S_FEAT, 1), lambda i: (0, 0)),
            pl.BlockSpec(memory_space=pltpu.MemorySpace.HBM),
            pl.BlockSpec((_S_FEAT, _LABELS), lambda i: (0, 0)),
            pl.BlockSpec((1, _LABELS), lambda i: (0, 0)),
        ],
        out_specs=[
            pl.BlockSpec(memory_space=pltpu.MemorySpace.HBM),
            pl.BlockSpec((_BR, _S_FEAT), lambda i: (i, 0)),
            pl.BlockSpec((1, _S_FEAT), lambda i: (0, 0)),
            pl.BlockSpec((1, _LABELS), lambda i: (0, 0)),
        ],
        out_shape=(
            jax.ShapeDtypeStruct((_N, _N), jnp.float32),
            jax.ShapeDtypeStruct((_N, _S_FEAT), jnp.float32),
            jax.ShapeDtypeStruct((1, _S_FEAT), jnp.float32),
            jax.ShapeDtypeStruct((1, _LABELS), jnp.float32),
        ),
        compiler_params=pltpu.CompilerParams(
            vmem_limit_bytes=67004416),
        scratch_shapes=[
            pltpu.VMEM((_N, _S_FEAT), jnp.bfloat16),
            pltpu.VMEM((_N, 1), jnp.float32),
            pltpu.VMEM((1, _N), jnp.float32),
            pltpu.VMEM((2, _BR, _N), jnp.int32),
            pltpu.SemaphoreType.DMA((2,)),
            pltpu.VMEM((_BR, _N), jnp.float32),
            pltpu.SemaphoreType.DMA((2,)),
        ],
    )(inDoc, W, a1, a2, adj, clsW, clsb2)

    return (pool.reshape(_S_FEAT), att, doc, label.reshape(_LABELS))


# speedup vs baseline: 1.1219x; 1.1219x over previous
import jax
import jax.numpy as jnp
from jax.experimental import pallas as pl
from jax.experimental.pallas import tpu as pltpu

_N, _IN_FEAT, _S_FEAT, _LABELS, _SLOPE = 4096, 512, 256, 2, 0.01
_BR = 512
_GRID = _N // _BR


def _adj_copy(adj_hbm, adj_buf, adj_sem, block, slot):
    return pltpu.make_async_copy(
        adj_hbm.at[pl.ds(block * _BR, _BR), :], adj_buf.at[slot],
        adj_sem.at[slot])


def _gat_kernel(inDoc_ref, W_ref, a1_ref, a2_ref, adj_hbm, clsW_ref, clsb_ref,
                att_ref, doc_ref, pool_ref, label_ref,
                whb_ref, f1_ref, f2t_ref, adj_buf, adj_sem):
    i = pl.program_id(0)
    slot = jax.lax.rem(i, 2)

    @pl.when(i == 0)
    def _():
        _adj_copy(adj_hbm, adj_buf, adj_sem, 0, 0).start()
        wa1 = jnp.dot(W_ref[...], a1_ref[...],
                      preferred_element_type=jnp.float32)
        wa2 = jnp.dot(W_ref[...], a2_ref[...],
                      preferred_element_type=jnp.float32)
        f1_ref[...] = jnp.dot(inDoc_ref[...], wa1,
                              preferred_element_type=jnp.float32)
        f2t_ref[...] = jax.lax.dot_general(
            wa2, inDoc_ref[...], (((0,), (1,)), ((), ())),
            preferred_element_type=jnp.float32)
        whb_ref[...] = jnp.dot(inDoc_ref[...], W_ref[...],
                               preferred_element_type=jnp.float32
                               ).astype(jnp.bfloat16)

    @pl.when(i + 1 < _GRID)
    def _():
        _adj_copy(adj_hbm, adj_buf, adj_sem, i + 1,
                  jax.lax.rem(i + 1, 2)).start()

    logits = f1_ref[pl.ds(i * _BR, _BR), :] + f2t_ref[...]
    logits = jnp.maximum(logits, _SLOPE * logits)
    q = jnp.exp(logits)

    _adj_copy(adj_hbm, adj_buf, adj_sem, i, slot).wait()
    p = jnp.where(adj_buf[slot] > 0, q, 0.0)
    pb = p.astype(jnp.bfloat16)
    recip = 1.0 / jnp.sum(p, axis=1, keepdims=True)
    att_ref[...] = p * recip
    doc = jnp.dot(pb, whb_ref[...],
                  preferred_element_type=jnp.float32) * recip
    doc = jnp.where(doc > 0, doc, jnp.exp(doc) - 1.0)
    doc_ref[...] = doc
    colsum = jnp.sum(doc, axis=0, keepdims=True)

    @pl.when(i == 0)
    def _():
        pool_ref[...] = colsum

    @pl.when(i > 0)
    def _():
        pool_ref[...] += colsum

    @pl.when(i == _GRID - 1)
    def _():
        pool = pool_ref[...] * (1.0 / _N)
        pool_ref[...] = pool
        cls = jnp.dot(pool, clsW_ref[...],
                      preferred_element_type=jnp.float32) + clsb_ref[...]
        cm = jnp.max(cls, axis=1, keepdims=True)
        cp = jnp.exp(cls - cm)
        label_ref[...] = cp / jnp.sum(cp, axis=1, keepdims=True)


def kernel(inDoc, adj, W, a1, a2, clsW, clsb):
    clsb2 = clsb.reshape(1, _LABELS)

    att, doc, pool, label = pl.pallas_call(
        _gat_kernel,
        grid=(_GRID,),
        in_specs=[
            pl.BlockSpec((_N, _IN_FEAT), lambda i: (0, 0)),
            pl.BlockSpec((_IN_FEAT, _S_FEAT), lambda i: (0, 0)),
            pl.BlockSpec((_S_FEAT, 1), lambda i: (0, 0)),
            pl.BlockSpec((_S_FEAT, 1), lambda i: (0, 0)),
            pl.BlockSpec(memory_space=pltpu.MemorySpace.HBM),
            pl.BlockSpec((_S_FEAT, _LABELS), lambda i: (0, 0)),
            pl.BlockSpec((1, _LABELS), lambda i: (0, 0)),
        ],
        out_specs=[
            pl.BlockSpec((_BR, _N), lambda i: (i, 0)),
            pl.BlockSpec((_BR, _S_FEAT), lambda i: (i, 0)),
            pl.BlockSpec((1, _S_FEAT), lambda i: (0, 0)),
            pl.BlockSpec((1, _LABELS), lambda i: (0, 0)),
        ],
        out_shape=(
            jax.ShapeDtypeStruct((_N, _N), jnp.float32),
            jax.ShapeDtypeStruct((_N, _S_FEAT), jnp.float32),
            jax.ShapeDtypeStruct((1, _S_FEAT), jnp.float32),
            jax.ShapeDtypeStruct((1, _LABELS), jnp.float32),
        ),
        scratch_shapes=[
            pltpu.VMEM((_N, _S_FEAT), jnp.bfloat16),
            pltpu.VMEM((_N, 1), jnp.float32),
            pltpu.VMEM((1, _N), jnp.float32),
            pltpu.VMEM((2, _BR, _N), jnp.int32),
            pltpu.SemaphoreType.DMA((2,)),
        ],
    )(inDoc, W, a1, a2, adj, clsW, clsb2)

    return (pool.reshape(_S_FEAT), att, doc, label.reshape(_LABELS))
